# baseline (speedup 1.0000x reference)
import jax
import jax.numpy as jnp
from jax.experimental import pallas as pl
from jax.experimental.pallas import tpu as pltpu

_N = 4096
_DP = 128
_D = 64
_RB = 1024
_NRB = _N // _RB
_P1 = 2
_P2 = 4
_EPI = (_P1 + _P2) * _NRB
_STEPS = _EPI + _NRB
_NBUF = 3
_DUMMY = 4


def _cell_kernel(ai_ref, rr_ref, fetch_ref, slot_ref, src_ref, dst_ref,
                 x_ref, w_ref, b_ref, g_ref, bt_ref, adj_ref, o_ref,
                 states_ref, buf_ref, sem_ref):
    n = pl.program_id(0)

    def _copy(m):
        return pltpu.make_async_copy(
            adj_ref.at[ai_ref[m], pl.ds(rr_ref[m] * _RB, _RB), :],
            buf_ref.at[slot_ref[m]],
            sem_ref.at[slot_ref[m]])

    @pl.when(n == 0)
    def _():
        for k in range(1, _NBUF - 1):
            @pl.when(fetch_ref[k] == 1)
            def _(k=k):
                _copy(k).start()

    @pl.when(n == 0)
    def _():
        _copy(0).start()
        h = jax.lax.dot_general(x_ref[...], w_ref[...],
                                (((1,), (1,)), ((), ())),
                                preferred_element_type=jnp.float32)
        states_ref[0] = (h + b_ref[0][None, :]).astype(jnp.bfloat16)
        states_ref[1] = jnp.zeros((_N, _D), jnp.bfloat16)
        states_ref[2] = jnp.zeros((_N, _D), jnp.bfloat16)
        states_ref[3] = jnp.zeros((_N, _D), jnp.bfloat16)

    m = jnp.minimum(n + _NBUF - 1, _STEPS - 1)

    @pl.when(jnp.logical_and(n + _NBUF - 1 < _STEPS, fetch_ref[m] == 1))
    def _():
        _copy(m).start()

    @pl.when(fetch_ref[n] == 1)
    def _():
        _copy(n).wait()

    row = pl.ds(rr_ref[n] * _RB, _RB)

    @pl.when(jnp.logical_and(n < _EPI, src_ref[n] != _DUMMY))
    def _():
        a = buf_ref[slot_ref[n]].astype(jnp.bfloat16)
        rhs = states_ref[src_ref[n]]
        contrib = jnp.dot(a, rhs, preferred_element_type=jnp.float32)
        states_ref[dst_ref[n], row] = (
            states_ref[dst_ref[n], row].astype(jnp.float32) + contrib
        ).astype(jnp.bfloat16)

    @pl.when(n >= _EPI)
    def _():
        a = buf_ref[slot_ref[n]].astype(jnp.bfloat16)
        s = states_ref[3, row].astype(jnp.float32) + jnp.dot(
            a, states_ref[2], preferred_element_type=jnp.float32)
        mu = jnp.mean(s, axis=-1, keepdims=True)
        var = jnp.mean((s - mu) ** 2, axis=-1, keepdims=True)
        ln = (s - mu) * jax.lax.rsqrt(var + 1e-5) * g_ref[0][None, :] \
            + bt_ref[0][None, :]
        o_ref[...] = 0.5 * ln * (1.0 + jax.lax.erf(ln * 0.7071067811865476))


def kernel(x, adjs, idxes_seq, idxes_res, W, b, gamma, beta):
    iseq = idxes_seq.astype(jnp.int32)
    ires = idxes_res.astype(jnp.int32)

    match0 = ires[0] == iseq[0]
    match1 = jnp.logical_and(ires[1] == iseq[0], jnp.logical_not(match0))
    taken = jnp.logical_or(match0, match1)
    rA_ai = jnp.where(taken, iseq[0], iseq[0])
    rA_src = jnp.where(taken, 0, _DUMMY)
    rA_dst = jnp.where(match0, 2, 3)

    b3_ai = jnp.where(match0, ires[1], ires[0])
    b3_dst = jnp.where(match0, 3, 2)
    b4_ai = jnp.where(taken, b3_ai, ires[1])
    b4_src = jnp.where(taken, _DUMMY, 0)
    p2_ai = jnp.stack([iseq[1], ires[2], b3_ai, b4_ai])
    p2_src = jnp.stack([jnp.int32(1), jnp.int32(1), jnp.int32(0), b4_src])
    p2_dst = jnp.stack([jnp.int32(2), jnp.int32(3), b3_dst, jnp.int32(3)])
    perm = jnp.argsort(p2_ai)
    p2_ai, p2_src, p2_dst = p2_ai[perm], p2_src[perm], p2_dst[perm]

    blk = jnp.arange(_NRB, dtype=jnp.int32)
    p1_ai = jnp.stack([iseq[0], rA_ai])
    p1_src = jnp.stack([jnp.int32(0), rA_src])
    p1_dst = jnp.stack([jnp.int32(1), rA_dst])

    ai_all = jnp.concatenate([
        jnp.tile(p1_ai, _NRB), jnp.tile(p2_ai, _NRB),
        jnp.full((_NRB,), iseq[2], jnp.int32)])
    rr_all = jnp.concatenate([
        jnp.repeat(blk, _P1), jnp.repeat(blk, _P2), blk])
    zpad = jnp.zeros((_NRB,), jnp.int32)
    src_all = jnp.concatenate([
        jnp.tile(p1_src, _NRB), jnp.tile(p2_src, _NRB), zpad])
    dst_all = jnp.concatenate([
        jnp.tile(p1_dst, _NRB), jnp.tile(p2_dst, _NRB), zpad])

    key = ai_all * _NRB + rr_all
    fetch = jnp.concatenate([
        jnp.ones((1,), jnp.int32), (key[1:] != key[:-1]).astype(jnp.int32)])
    slot = jnp.mod(jnp.cumsum(fetch) - 1, _NBUF).astype(jnp.int32)

    grid_spec = pltpu.PrefetchScalarGridSpec(
        num_scalar_prefetch=6,
        grid=(_STEPS,),
        in_specs=[
            pl.BlockSpec((_N, _DP), lambda n, *s: (0, 0)),
            pl.BlockSpec((_D, _DP), lambda n, *s: (0, 0)),
            pl.BlockSpec((1, _D), lambda n, *s: (0, 0)),
            pl.BlockSpec((1, _D), lambda n, *s: (0, 0)),
            pl.BlockSpec((1, _D), lambda n, *s: (0, 0)),
            pl.BlockSpec(memory_space=pltpu.MemorySpace.HBM),
        ],
        out_specs=pl.BlockSpec(
            (_RB, _D),
            lambda n, *s: (jnp.where(n >= _EPI, n - _EPI, 0), 0)),
        scratch_shapes=[
            pltpu.VMEM((4, _N, _D), jnp.bfloat16),
            pltpu.VMEM((_NBUF, _RB, _N), jnp.float32),
            pltpu.SemaphoreType.DMA((_NBUF,)),
        ],
    )
    return pl.pallas_call(
        _cell_kernel,
        grid_spec=grid_spec,
        out_shape=jax.ShapeDtypeStruct((_N, _D), jnp.float32),
        compiler_params=pltpu.CompilerParams(
            vmem_limit_bytes=100 * 1024 * 1024),
    )(ai_all, rr_all, fetch, slot, src_all, dst_all, x.astype(jnp.bfloat16),
      W.astype(jnp.bfloat16), b.reshape(1, _D),
      gamma.reshape(1, _D), beta.reshape(1, _D), adjs)


# speedup vs baseline: 1.0991x; 1.0991x over previous
import jax
import jax.numpy as jnp
from jax.experimental import pallas as pl
from jax.experimental.pallas import tpu as pltpu

_N = 4096
_DP = 128
_D = 64
_RB = 512
_NRB = _N // _RB
_P1 = 2
_P2 = 4
_EPI = (_P1 + _P2) * _NRB
_STEPS = _EPI + _NRB
_NBUF = 5
_DUMMY = 4


def _cell_kernel(ai_ref, rr_ref, fetch_ref, slot_ref, src_ref, dst_ref,
                 x_ref, w_ref, b_ref, g_ref, bt_ref, adj_ref, o_ref,
                 states_ref, buf_ref, sem_ref):
    n = pl.program_id(0)

    def _copy(m):
        return pltpu.make_async_copy(
            adj_ref.at[ai_ref[m], pl.ds(rr_ref[m] * _RB, _RB), :],
            buf_ref.at[slot_ref[m]],
            sem_ref.at[slot_ref[m]])

    @pl.when(n == 0)
    def _():
        for k in range(1, _NBUF - 1):
            @pl.when(fetch_ref[k] == 1)
            def _(k=k):
                _copy(k).start()

    @pl.when(n == 0)
    def _():
        _copy(0).start()
        h = jax.lax.dot_general(x_ref[...], w_ref[...],
                                (((1,), (1,)), ((), ())),
                                preferred_element_type=jnp.float32)
        states_ref[0] = h + b_ref[0][None, :]
        states_ref[1] = jnp.zeros((_N, _D), jnp.float32)
        states_ref[2] = jnp.zeros((_N, _D), jnp.float32)
        states_ref[3] = jnp.zeros((_N, _D), jnp.float32)

    m = jnp.minimum(n + _NBUF - 1, _STEPS - 1)

    @pl.when(jnp.logical_and(n + _NBUF - 1 < _STEPS, fetch_ref[m] == 1))
    def _():
        _copy(m).start()

    @pl.when(fetch_ref[n] == 1)
    def _():
        _copy(n).wait()

    row = pl.ds(rr_ref[n] * _RB, _RB)

    @pl.when(jnp.logical_and(n < _EPI, src_ref[n] != _DUMMY))
    def _():
        a = buf_ref[slot_ref[n]].astype(jnp.bfloat16)
        rhs = states_ref[src_ref[n]].astype(jnp.bfloat16)
        contrib = jnp.dot(a, rhs, preferred_element_type=jnp.float32)
        states_ref[dst_ref[n], row] += contrib

    @pl.when(n >= _EPI)
    def _():
        a = buf_ref[slot_ref[n]].astype(jnp.bfloat16)
        s = states_ref[3, row] + jnp.dot(
            a, states_ref[2].astype(jnp.bfloat16),
            preferred_element_type=jnp.float32)
        mu = jnp.mean(s, axis=-1, keepdims=True)
        var = jnp.mean((s - mu) ** 2, axis=-1, keepdims=True)
        ln = (s - mu) * jax.lax.rsqrt(var + 1e-5) * g_ref[0][None, :] \
            + bt_ref[0][None, :]
        o_ref[...] = 0.5 * ln * (1.0 + jax.lax.erf(ln * 0.7071067811865476))


def kernel(x, adjs, idxes_seq, idxes_res, W, b, gamma, beta):
    iseq = idxes_seq.astype(jnp.int32)
    ires = idxes_res.astype(jnp.int32)

    match0 = ires[0] == iseq[0]
    match1 = jnp.logical_and(ires[1] == iseq[0], jnp.logical_not(match0))
    taken = jnp.logical_or(match0, match1)
    rA_ai = jnp.where(taken, iseq[0], iseq[0])
    rA_src = jnp.where(taken, 0, _DUMMY)
    rA_dst = jnp.where(match0, 2, 3)

    b3_ai = jnp.where(match0, ires[1], ires[0])
    b3_dst = jnp.where(match0, 3, 2)
    b4_ai = jnp.where(taken, b3_ai, ires[1])
    b4_src = jnp.where(taken, _DUMMY, 0)
    p2_ai = jnp.stack([iseq[1], ires[2], b3_ai, b4_ai])
    p2_src = jnp.stack([jnp.int32(1), jnp.int32(1), jnp.int32(0), b4_src])
    p2_dst = jnp.stack([jnp.int32(2), jnp.int32(3), b3_dst, jnp.int32(3)])
    perm = jnp.argsort(p2_ai)
    p2_ai, p2_src, p2_dst = p2_ai[perm], p2_src[perm], p2_dst[perm]

    blk = jnp.arange(_NRB, dtype=jnp.int32)
    p1_ai = jnp.stack([iseq[0], rA_ai])
    p1_src = jnp.stack([jnp.int32(0), rA_src])
    p1_dst = jnp.stack([jnp.int32(1), rA_dst])

    ai_all = jnp.concatenate([
        jnp.tile(p1_ai, _NRB), jnp.tile(p2_ai, _NRB),
        jnp.full((_NRB,), iseq[2], jnp.int32)])
    rr_all = jnp.concatenate([
        jnp.repeat(blk, _P1), jnp.repeat(blk, _P2), blk])
    zpad = jnp.zeros((_NRB,), jnp.int32)
    src_all = jnp.concatenate([
        jnp.tile(p1_src, _NRB), jnp.tile(p2_src, _NRB), zpad])
    dst_all = jnp.concatenate([
        jnp.tile(p1_dst, _NRB), jnp.tile(p2_dst, _NRB), zpad])

    key = ai_all * _NRB + rr_all
    fetch = jnp.concatenate([
        jnp.ones((1,), jnp.int32), (key[1:] != key[:-1]).astype(jnp.int32)])
    slot = jnp.mod(jnp.cumsum(fetch) - 1, _NBUF).astype(jnp.int32)

    grid_spec = pltpu.PrefetchScalarGridSpec(
        num_scalar_prefetch=6,
        grid=(_STEPS,),
        in_specs=[
            pl.BlockSpec((_N, _DP), lambda n, *s: (0, 0)),
            pl.BlockSpec((_D, _DP), lambda n, *s: (0, 0)),
            pl.BlockSpec((1, _D), lambda n, *s: (0, 0)),
            pl.BlockSpec((1, _D), lambda n, *s: (0, 0)),
            pl.BlockSpec((1, _D), lambda n, *s: (0, 0)),
            pl.BlockSpec(memory_space=pltpu.MemorySpace.HBM),
        ],
        out_specs=pl.BlockSpec(
            (_RB, _D),
            lambda n, *s: (jnp.where(n >= _EPI, n - _EPI, 0), 0)),
        scratch_shapes=[
            pltpu.VMEM((4, _N, _D), jnp.float32),
            pltpu.VMEM((_NBUF, _RB, _N), jnp.float32),
            pltpu.SemaphoreType.DMA((_NBUF,)),
        ],
    )
    return pl.pallas_call(
        _cell_kernel,
        grid_spec=grid_spec,
        out_shape=jax.ShapeDtypeStruct((_N, _D), jnp.float32),
        compiler_params=pltpu.CompilerParams(
            vmem_limit_bytes=100 * 1024 * 1024),
    )(ai_all, rr_all, fetch, slot, src_all, dst_all, x, W, b.reshape(1, _D),
      gamma.reshape(1, _D), beta.reshape(1, _D), adjs)
